# Initial kernel scaffold; baseline (speedup 1.0000x reference)
import functools

import jax
import jax.numpy as jnp
from jax import lax
from jax.experimental import pallas as pl
from jax.experimental.pallas import tpu as pltpu
from jax.experimental.pallas import tpu_sc as plsc

B = 16384
NF = 26
V = 100000
D = 32
L = 50
NUMF = 16

NC = 2
NS = 16
NW = NC * NS
RPW = B // NW
BAG_C = 32
BAG_NCH = RPW // BAG_C


def _sc_body(catT, sidx, nff, tab, emb, w, out,
             cidx0, cidx1, cdat0, cdat1, sidx_v, sdat_v, bag_v,
             nf_v, num_v, w_v, sem0, sem1, semb):
    wid = lax.axis_index("s") * NC + lax.axis_index("c")
    base = wid * RPW

    cidx = [cidx0, cidx1]
    cdat = [cdat0, cdat1]
    sems = [sem0, sem1]
    cps = [None, None]
    pltpu.sync_copy(catT.at[0, pl.ds(base, RPW)], cidx[0])
    cps[0] = pltpu.async_copy(tab.at[cidx[0]], cdat[0], sems[0])
    for f in range(1, NF + 1):
        if f < NF:
            pltpu.sync_copy(catT.at[f, pl.ds(base, RPW)], cidx[f % 2])
            cps[f % 2] = pltpu.async_copy(tab.at[cidx[f % 2]], cdat[f % 2],
                                          sems[f % 2])
        cps[(f - 1) % 2].wait()
        pltpu.sync_copy(cdat[(f - 1) % 2], out.at[pl.ds(base, RPW), f - 1])

    lane = lax.iota(jnp.int32, (16,))
    mhi = lane >= 14

    def bag_row(b, _):
        rb = b * L
        a0 = jnp.zeros((16,), jnp.float32)
        a1 = jnp.zeros((16,), jnp.float32)
        a2 = jnp.zeros((16,), jnp.float32)
        a3 = jnp.zeros((16,), jnp.float32)
        for l in range(0, L, 2):
            a0 = a0 + sdat_v[rb + l, 0:16]
            a1 = a1 + sdat_v[rb + l, 16:32]
            a2 = a2 + sdat_v[rb + l + 1, 0:16]
            a3 = a3 + sdat_v[rb + l + 1, 16:32]
        s0 = a0 + a2
        s1 = a1 + a3
        m0 = sidx_v[pl.ds(rb, 16)] != 0
        m1 = sidx_v[pl.ds(rb + 16, 16)] != 0
        m2 = sidx_v[pl.ds(rb + 32, 16)] != 0
        m3 = (sidx_v[pl.ds(rb + 34, 16)] != 0) & mhi
        cnt = (plsc.all_reduce_population_count(m0)
               + plsc.all_reduce_population_count(m1)
               + plsc.all_reduce_population_count(m2)
               + plsc.all_reduce_population_count(m3))
        scale = 1.0 / jnp.maximum(cnt.astype(jnp.float32), 1.0)
        bag_v[b, 0:16] = s0 * scale
        bag_v[b, 16:32] = s1 * scale
        return 0

    def bag_chunk(c, _):
        gb = base + c * BAG_C
        pltpu.sync_copy(sidx.at[pl.ds(gb * L, BAG_C * L)], sidx_v)
        pltpu.async_copy(emb.at[sidx_v], sdat_v, semb).wait()
        lax.fori_loop(0, BAG_C, bag_row, 0)
        pltpu.sync_copy(bag_v, out.at[pl.ds(gb, BAG_C), NF])
        return 0

    lax.fori_loop(0, BAG_NCH, bag_chunk, 0)

    pltpu.sync_copy(w, w_v)
    pltpu.sync_copy(nff.at[pl.ds(base * NUMF, RPW * NUMF)], nf_v)
    wr0 = [w_v[k, 0:16] for k in range(NUMF)]
    wr1 = [w_v[k, 16:32] for k in range(NUMF)]

    def num_row(r, _):
        acc0 = jnp.zeros((16,), jnp.float32)
        acc1 = jnp.zeros((16,), jnp.float32)
        rb = r * NUMF
        for k in range(NUMF):
            s = plsc.load_gather(nf_v, [jnp.full((16,), rb + k, jnp.int32)])
            acc0 = acc0 + s * wr0[k]
            acc1 = acc1 + s * wr1[k]
        num_v[r, 0:16] = acc0
        num_v[r, 16:32] = acc1
        return 0

    lax.fori_loop(0, RPW, num_row, 0)
    pltpu.sync_copy(num_v, out.at[pl.ds(base, RPW), NF + 1])


_sc_kernel = functools.partial(
    pl.kernel,
    out_type=jax.ShapeDtypeStruct((B, NF + 2, D), jnp.float32),
    mesh=plsc.VectorSubcoreMesh(core_axis_name="c", subcore_axis_name="s"),
    scratch_types=[
        pltpu.VMEM((RPW,), jnp.int32),
        pltpu.VMEM((RPW,), jnp.int32),
        pltpu.VMEM((RPW, D), jnp.float32),
        pltpu.VMEM((RPW, D), jnp.float32),
        pltpu.VMEM((BAG_C * L,), jnp.int32),
        pltpu.VMEM((BAG_C * L, D), jnp.float32),
        pltpu.VMEM((BAG_C, D), jnp.float32),
        pltpu.VMEM((RPW * NUMF,), jnp.float32),
        pltpu.VMEM((RPW, D), jnp.float32),
        pltpu.VMEM((NUMF, D), jnp.float32),
        pltpu.SemaphoreType.DMA,
        pltpu.SemaphoreType.DMA,
        pltpu.SemaphoreType.DMA,
    ],
)(_sc_body)


def kernel(cat_indices, seq_indices, num_feat, tables_cat, emb_seq, W_num):
    offs = jnp.arange(NF, dtype=jnp.int32) * V
    catT = (cat_indices + offs[None, :]).T
    sflat = seq_indices.reshape(B * L)
    nff = num_feat.reshape(B * NUMF)
    tab = tables_cat.reshape(NF * V, D)
    out3 = _sc_kernel(catT, sflat, nff, tab, emb_seq, W_num)
    return out3.reshape(B, (NF + 2) * D)


# speedup vs baseline: 2.3383x; 2.3383x over previous
import functools

import jax
import jax.numpy as jnp
from jax import lax
from jax.experimental import pallas as pl
from jax.experimental.pallas import tpu as pltpu
from jax.experimental.pallas import tpu_sc as plsc

B = 16384
NF = 26
V = 100000
D = 32
L = 50
NUMF = 16

NC = 2
NS = 16
NW = NC * NS
RPW = B // NW
BAG_C = 32
BAG_NCH = RPW // BAG_C


def _sc_body(catT, sidx, nff, tab, emb, w, out,
             cidx0, cidx1, cdat0, cdat1, sidx_v, sdat_v, bag_v,
             nf_v, num_v, w_v, sem0, sem1, semb):
    wid = lax.axis_index("s") * NC + lax.axis_index("c")
    base = wid * RPW

    cidx = [cidx0, cidx1]
    cdat = [cdat0, cdat1]
    sems = [sem0, sem1]
    cps = [None, None]
    pltpu.sync_copy(catT.at[0, pl.ds(base, RPW)], cidx[0])
    cps[0] = pltpu.async_copy(tab.at[cidx[0]], cdat[0], sems[0])
    for f in range(1, NF + 1):
        if f < NF:
            pltpu.sync_copy(catT.at[f, pl.ds(base, RPW)], cidx[f % 2])
            cps[f % 2] = pltpu.async_copy(tab.at[cidx[f % 2]], cdat[f % 2],
                                          sems[f % 2])
        cps[(f - 1) % 2].wait()
        pltpu.sync_copy(cdat[(f - 1) % 2], out.at[pl.ds(base, RPW), f - 1])

    lane = lax.iota(jnp.int32, 16)
    mhi = lane >= 14

    def bag_row(b, _):
        rb = b * L
        a0 = jnp.zeros((16,), jnp.float32)
        a1 = jnp.zeros((16,), jnp.float32)
        a2 = jnp.zeros((16,), jnp.float32)
        a3 = jnp.zeros((16,), jnp.float32)
        for l in range(0, L, 2):
            a0 = a0 + sdat_v[rb + l, 0:16]
            a1 = a1 + sdat_v[rb + l, 16:32]
            a2 = a2 + sdat_v[rb + l + 1, 0:16]
            a3 = a3 + sdat_v[rb + l + 1, 16:32]
        s0 = a0 + a2
        s1 = a1 + a3
        m0 = sidx_v[pl.ds(rb, 16)] != 0
        m1 = sidx_v[pl.ds(rb + 16, 16)] != 0
        m2 = sidx_v[pl.ds(rb + 32, 16)] != 0
        m3 = (sidx_v[pl.ds(rb + 34, 16)] != 0) & mhi
        cnt = (plsc.all_reduce_population_count(m0)
               + plsc.all_reduce_population_count(m1)
               + plsc.all_reduce_population_count(m2)
               + plsc.all_reduce_population_count(m3))
        scale = 1.0 / jnp.maximum(cnt.astype(jnp.float32), 1.0)
        bag_v[b, 0:16] = s0 * scale
        bag_v[b, 16:32] = s1 * scale
        return 0

    def bag_chunk(c, _):
        gb = base + c * BAG_C
        pltpu.sync_copy(sidx.at[pl.ds(gb * L, BAG_C * L)], sidx_v)
        pltpu.async_copy(emb.at[sidx_v], sdat_v, semb).wait()
        lax.fori_loop(0, BAG_C, bag_row, 0)
        pltpu.sync_copy(bag_v, out.at[pl.ds(gb, BAG_C), NF])
        return 0

    lax.fori_loop(0, BAG_NCH, bag_chunk, 0)

    pltpu.sync_copy(w, w_v)
    pltpu.sync_copy(nff.at[pl.ds(base * NUMF, RPW * NUMF)], nf_v)
    wr0 = [w_v[k, 0:16] for k in range(NUMF)]
    wr1 = [w_v[k, 16:32] for k in range(NUMF)]

    def num_row(r, _):
        acc0 = jnp.zeros((16,), jnp.float32)
        acc1 = jnp.zeros((16,), jnp.float32)
        rb = r * NUMF
        for k in range(NUMF):
            s = plsc.load_gather(nf_v, [jnp.full((16,), rb + k, jnp.int32)])
            acc0 = acc0 + s * wr0[k]
            acc1 = acc1 + s * wr1[k]
        num_v[r, 0:16] = acc0
        num_v[r, 16:32] = acc1
        return 0

    lax.fori_loop(0, RPW, num_row, 0)
    pltpu.sync_copy(num_v, out.at[pl.ds(base, RPW), NF + 1])


_sc_kernel = functools.partial(
    pl.kernel,
    out_type=jax.ShapeDtypeStruct((B, NF + 2, D), jnp.float32),
    mesh=plsc.VectorSubcoreMesh(core_axis_name="c", subcore_axis_name="s"),
    compiler_params=pltpu.CompilerParams(needs_layout_passes=False,
                                         use_tc_tiling_on_sc=False),
    scratch_types=[
        pltpu.VMEM((RPW,), jnp.int32),
        pltpu.VMEM((RPW,), jnp.int32),
        pltpu.VMEM((RPW, D), jnp.float32),
        pltpu.VMEM((RPW, D), jnp.float32),
        pltpu.VMEM((BAG_C * L,), jnp.int32),
        pltpu.VMEM((BAG_C * L, D), jnp.float32),
        pltpu.VMEM((BAG_C, D), jnp.float32),
        pltpu.VMEM((RPW * NUMF,), jnp.float32),
        pltpu.VMEM((RPW, D), jnp.float32),
        pltpu.VMEM((NUMF, D), jnp.float32),
        pltpu.SemaphoreType.DMA,
        pltpu.SemaphoreType.DMA,
        pltpu.SemaphoreType.DMA,
    ],
)(_sc_body)


def kernel(cat_indices, seq_indices, num_feat, tables_cat, emb_seq, W_num):
    offs = jnp.arange(NF, dtype=jnp.int32) * V
    catT = (cat_indices + offs[None, :]).T
    sflat = seq_indices.reshape(B * L)
    nff = num_feat.reshape(B * NUMF)
    tab = tables_cat.reshape(NF * V, D)
    out3 = _sc_kernel(catT, sflat, nff, tab, emb_seq, W_num)
    return out3.reshape(B, (NF + 2) * D)
